# Initial kernel scaffold; baseline (speedup 1.0000x reference)
import jax
import jax.numpy as jnp
from jax.experimental import pallas as pl


def kernel(x, table):
    raise NotImplementedError("write your pallas kernel here")


# speedup vs baseline: 8.1418x; 8.1418x over previous
import functools

import jax
import jax.numpy as jnp
from jax import lax
from jax.experimental import pallas as pl
from jax.experimental.pallas import tpu as pltpu
from jax.experimental.pallas import tpu_sc as plsc

VOCAB = 50257
DIM = 128
B_TOKENS = 4096 * 200
NC, NS = 2, 16
NW = NC * NS
B_PER_W = B_TOKENS // NW
CHUNK = 512
GATHER = 128
STEPS = B_PER_W // CHUNK
N_GATHERS = CHUNK // GATHER


@functools.partial(
    pl.kernel,
    out_type=jax.ShapeDtypeStruct((B_TOKENS, DIM), jnp.float32),
    mesh=plsc.VectorSubcoreMesh(core_axis_name="c", subcore_axis_name="s"),
    scratch_types=[
        pltpu.VMEM((CHUNK,), jnp.int32),
        pltpu.VMEM((CHUNK, DIM), jnp.float32),
        pltpu.SemaphoreType.DMA,
    ],
)
def _gather_kernel(table_hbm, idx_hbm, out_hbm, idx_v, rows_v, sem):
    wid = lax.axis_index("s") * NC + lax.axis_index("c")
    base = wid * B_PER_W

    def step(s, _):
        off = base + s * CHUNK
        pltpu.sync_copy(idx_hbm.at[pl.ds(off, CHUNK)], idx_v)
        copies = []
        for j in range(N_GATHERS):
            copies.append(pltpu.async_copy(
                table_hbm.at[idx_v.at[pl.ds(j * GATHER, GATHER)]],
                rows_v.at[pl.ds(j * GATHER, GATHER)],
                sem,
            ))
        for c in copies:
            c.wait()
        pltpu.sync_copy(rows_v, out_hbm.at[pl.ds(off, CHUNK)])
        return ()

    lax.fori_loop(0, STEPS, step, (), unroll=False)


def kernel(x, table):
    idx = x.reshape(-1).astype(jnp.int32)
    out = _gather_kernel(table, idx)
    return out.reshape(x.shape[0], x.shape[1], DIM)


# speedup vs baseline: 9.2067x; 1.1308x over previous
import functools

import jax
import jax.numpy as jnp
from jax import lax
from jax.experimental import pallas as pl
from jax.experimental.pallas import tpu as pltpu
from jax.experimental.pallas import tpu_sc as plsc

VOCAB = 50257
DIM = 128
B_TOKENS = 4096 * 200
NC, NS = 2, 16
NW = NC * NS
B_PER_W = B_TOKENS // NW
CHUNK = 128
STEPS = B_PER_W // CHUNK
NBUF = 4
DIST = 2


@functools.partial(
    pl.kernel,
    out_type=jax.ShapeDtypeStruct((B_TOKENS, DIM), jnp.float32),
    mesh=plsc.VectorSubcoreMesh(core_axis_name="c", subcore_axis_name="s"),
    scratch_types=[
        pltpu.VMEM((B_PER_W,), jnp.int32),
        pltpu.VMEM((NBUF, CHUNK, DIM), jnp.float32),
    ] + [pltpu.SemaphoreType.DMA] * (2 * NBUF),
)
def _gather_kernel(table_hbm, idx_hbm, out_hbm, idx_v, rows_v, *sems):
    gsem, wsem = sems[:NBUF], sems[NBUF:]
    wid = lax.axis_index("s") * NC + lax.axis_index("c")
    base = wid * B_PER_W
    pltpu.sync_copy(idx_hbm.at[pl.ds(base, B_PER_W)], idx_v)

    def gather_desc(s, b):
        return pltpu.make_async_copy(
            table_hbm.at[idx_v.at[pl.ds(s * CHUNK, CHUNK)]],
            rows_v.at[b], gsem[b])

    def write_desc(s, b):
        return pltpu.make_async_copy(
            rows_v.at[b], out_hbm.at[pl.ds(base + s * CHUNK, CHUNK)], wsem[b])

    def step(s, b, first, last):
        if not last:
            bp = (b + DIST) % NBUF
            if not first:
                write_desc(s - DIST, bp).wait()
            gather_desc(s + DIST, bp).start()
        gather_desc(s, b).wait()
        write_desc(s, b).start()

    for s in range(DIST):
        gather_desc(s, s).start()
    for b in range(NBUF):
        step(b, b, first=(b < DIST), last=False)

    def group(g, _):
        for b in range(NBUF):
            step(NBUF * g + b, b, first=False, last=False)
        return ()

    n_groups = STEPS // NBUF
    lax.fori_loop(1, n_groups - 1, group, (), unroll=False)

    for b in range(NBUF):
        step(NBUF * (n_groups - 1) + b, b, first=False, last=(b >= NBUF - DIST))
    for b in range(NBUF):
        write_desc(STEPS - NBUF + b, b).wait()


def kernel(x, table):
    idx = x.reshape(-1).astype(jnp.int32)
    out = _gather_kernel(table, idx)
    return out.reshape(x.shape[0], x.shape[1], DIM)
